# Initial kernel scaffold; baseline (speedup 1.0000x reference)
import jax
import jax.numpy as jnp
from jax.experimental import pallas as pl


def kernel(x, src0, dst0, src1, dst1, W_self1, W_neigh1, b1, W_self2, W_neigh2, b2):
    raise NotImplementedError("write your pallas kernel here")


# speedup vs baseline: 1.1015x; 1.1015x over previous
import jax
import jax.numpy as jnp
from jax import lax
from jax.experimental import pallas as pl
from jax.experimental.pallas import tpu as pltpu
from jax.experimental.pallas import tpu_sc as plsc

F32 = jnp.float32
I32 = jnp.int32

NUM_SRC0 = 10000
NUM_DST0 = 5000
NUM_DST1 = 2000
E0 = 160000
E1 = 64000
IN_FEATS = 256
H_FEATS = 256
NUM_CLASSES = 64

NC = 2
NS = 16
NW = NC * NS


def _make_sc_seg_sum(D, E, N, K, DSPLIT):
    CB = D // 16
    EG = NW // CB
    assert CB * EG == NW and E % EG == 0
    n_per = E // EG
    assert n_per % K == 0 and K % 8 == 0 and K <= 128
    nchunks = n_per // K
    assert N % DSPLIT == 0 and DSPLIT <= CB
    ND = N // DSPLIT

    def body(table_hbm, src_hbm, dst_hbm, sums_out, degs_out,
             acc, dacc, sidx, didx, rows, gsem):
        ci = lax.axis_index("c")
        si = lax.axis_index("s")
        wid = si * NC + ci
        eg = wid // CB
        cb = wid % CB
        base = eg * n_per
        z16 = jnp.zeros((16,), F32)
        o16 = jnp.ones((16,), F32)
        zi16 = jnp.zeros((16,), I32)
        iota16 = lax.iota(I32, 16)

        def _zacc(r, c):
            acc[r, pl.ds(0, 16)] = z16
            return c
        lax.fori_loop(0, N, _zacc, 0)

        @pl.when(cb < DSPLIT)
        def _():
            def _zdeg(r, c):
                dacc[r, pl.ds(0, 16)] = z16
                return c
            lax.fori_loop(0, ND, _zdeg, 0)

        dlo = cb * ND

        def _chunk(i, c):
            off = base + i * K
            pltpu.sync_copy(src_hbm.at[pl.ds(off, K)], sidx)
            pltpu.sync_copy(dst_hbm.at[pl.ds(off, K)], didx)
            pltpu.async_copy(table_hbm.at[cb].at[sidx], rows, gsem).wait()
            for g in range(K // 16):
                dvec = didx[pl.ds(g * 16, 16)]
                for e in range(16):
                    sel = jnp.full((16,), e, I32)
                    rowv = jnp.take_along_axis(dvec, sel, axis=0)
                    vals = rows[g * 16 + e, pl.ds(0, 16)]
                    plsc.addupdate_scatter(acc, [rowv, iota16], vals)

            @pl.when(cb < DSPLIT)
            def _():
                for g in range(K // 16):
                    dvec = didx[pl.ds(g * 16, 16)]
                    m = (dvec >= dlo) & (dvec < dlo + ND)
                    loc = jnp.where(m, dvec - dlo, 0)
                    plsc.addupdate_scatter(dacc, [loc, zi16], o16, mask=m)
            return c

        lax.fori_loop(0, nchunks, _chunk, 0)

        pltpu.sync_copy(acc, sums_out.at[eg, :, pl.ds(cb * 16, 16)])

        @pl.when(cb < DSPLIT)
        def _():
            pltpu.sync_copy(dacc, degs_out.at[eg, cb])

    mesh = plsc.VectorSubcoreMesh(core_axis_name="c", subcore_axis_name="s")
    return pl.kernel(
        body,
        out_type=(jax.ShapeDtypeStruct((EG, N, D), F32),
                  jax.ShapeDtypeStruct((EG, DSPLIT, ND, 16), F32)),
        mesh=mesh,
        compiler_params=pltpu.CompilerParams(
            use_tc_tiling_on_sc=False, needs_layout_passes=False),
        scratch_types=[
            pltpu.VMEM((N, 16), F32),
            pltpu.VMEM((ND, 16), F32),
            pltpu.VMEM((K,), I32),
            pltpu.VMEM((K,), I32),
            pltpu.VMEM((K, 16), F32),
            pltpu.SemaphoreType.DMA,
        ],
    )


_sc_seg1 = _make_sc_seg_sum(IN_FEATS, E0, NUM_DST0, K=128, DSPLIT=2)
_sc_seg2 = _make_sc_seg_sum(NUM_CLASSES, E1, NUM_DST1, K=80, DSPLIT=4)


def _tc1_body(x_ref, s_ref, d_ref, ws1_ref, wn1_ref, b1_ref, wn2_ref,
              ws2_ref, p_ref, hs_ref):
    s = s_ref[0] + s_ref[1]
    deg = d_ref[0, :, :1] + d_ref[1, :, :1]
    inv = jnp.where(deg > 0.0, 1.0 / jnp.maximum(deg, 1.0), 0.0)
    mean = s * inv
    h = jnp.dot(x_ref[...], ws1_ref[...], preferred_element_type=F32)
    h = h + jnp.dot(mean, wn1_ref[...], preferred_element_type=F32)
    h = jnp.maximum(h + b1_ref[...], 0.0)
    p_ref[...] = jnp.dot(h, wn2_ref[...], preferred_element_type=F32)
    hs_ref[...] = jnp.dot(h, ws2_ref[...], preferred_element_type=F32)


_TC1_BM = 1000

_tc1 = pl.pallas_call(
    _tc1_body,
    grid=(NUM_DST0 // _TC1_BM,),
    in_specs=[
        pl.BlockSpec((_TC1_BM, IN_FEATS), lambda i: (i, 0)),
        pl.BlockSpec((2, _TC1_BM, IN_FEATS), lambda i: (0, i, 0)),
        pl.BlockSpec((2, _TC1_BM, 16), lambda i: (0, i, 0)),
        pl.BlockSpec((IN_FEATS, H_FEATS), lambda i: (0, 0)),
        pl.BlockSpec((IN_FEATS, H_FEATS), lambda i: (0, 0)),
        pl.BlockSpec((1, H_FEATS), lambda i: (0, 0)),
        pl.BlockSpec((H_FEATS, NUM_CLASSES), lambda i: (0, 0)),
        pl.BlockSpec((H_FEATS, NUM_CLASSES), lambda i: (0, 0)),
    ],
    out_specs=[
        pl.BlockSpec((_TC1_BM, NUM_CLASSES), lambda i: (i, 0)),
        pl.BlockSpec((_TC1_BM, NUM_CLASSES), lambda i: (i, 0)),
    ],
    out_shape=[
        jax.ShapeDtypeStruct((NUM_DST0, NUM_CLASSES), F32),
        jax.ShapeDtypeStruct((NUM_DST0, NUM_CLASSES), F32),
    ],
)


def _tc2_body(hs_ref, s_ref, d_ref, b2_ref, o_ref):
    s = jnp.sum(s_ref[...], axis=0)
    deg = jnp.sum(d_ref[...], axis=0)[:, :1]
    inv = jnp.where(deg > 0.0, 1.0 / jnp.maximum(deg, 1.0), 0.0)
    o_ref[...] = hs_ref[...] + s * inv + b2_ref[...]


_tc2 = pl.pallas_call(
    _tc2_body,
    out_shape=jax.ShapeDtypeStruct((NUM_DST1, NUM_CLASSES), F32),
)


def kernel(x, src0, dst0, src1, dst1, W_self1, W_neigh1, b1,
           W_self2, W_neigh2, b2):
    xt = x.reshape(NUM_SRC0, IN_FEATS // 16, 16).transpose(1, 0, 2)
    sums0, degs0 = _sc_seg1(xt, src0, dst0)
    degs0 = degs0.reshape(2, NUM_DST0, 16)
    p, hsall = _tc1(x[:NUM_DST0], sums0, degs0, W_self1, W_neigh1,
                    b1.reshape(1, -1), W_neigh2, W_self2)
    pt = p.reshape(NUM_DST0, NUM_CLASSES // 16, 16).transpose(1, 0, 2)
    sums1, degs1 = _sc_seg2(pt, src1, dst1)
    degs1 = degs1.reshape(8, NUM_DST1, 16)
    out = _tc2(hsall[:NUM_DST1], sums1, degs1, b2.reshape(1, -1))
    return out


# speedup vs baseline: 2.1014x; 1.9077x over previous
import jax
import jax.numpy as jnp
from jax import lax
from jax.experimental import pallas as pl
from jax.experimental.pallas import tpu as pltpu
from jax.experimental.pallas import tpu_sc as plsc

F32 = jnp.float32
I32 = jnp.int32

NUM_SRC0 = 10000
NUM_DST0 = 5000
NUM_DST1 = 2000
E0 = 160000
E1 = 64000
IN_FEATS = 256
H_FEATS = 256
NUM_CLASSES = 64

NC = 2
NS = 16
NW = NC * NS


def _make_sc_seg_sum(D, E, N, K, G, DSPLIT):
    CB = D // 16
    EG = NW // CB
    assert CB * EG == NW and E % EG == 0
    n_per = E // EG
    M = K * G
    assert n_per % M == 0 and K % 8 == 0 and K <= 128
    nsup = n_per // M
    assert nsup % 2 == 0 and nsup >= 4
    nj = nsup // 2
    assert N % DSPLIT == 0 and DSPLIT <= CB
    ND = N // DSPLIT

    def body(table_hbm, src_hbm, dst_hbm, sums_out, degs_out,
             acc, dacc, sidx0, didx0, rows0, sidx1, didx1, rows1,
             gsem0, gsem1, isem0, isem1):
        ci = lax.axis_index("c")
        si = lax.axis_index("s")
        wid = si * NC + ci
        eg = wid // CB
        cb = wid % CB
        base = eg * n_per
        z16 = jnp.zeros((16,), F32)
        o16 = jnp.ones((16,), F32)
        zi16 = jnp.zeros((16,), I32)
        iota16 = lax.iota(I32, 16)
        dlo = cb * ND

        bufs = ((sidx0, didx0, rows0, gsem0, isem0),
                (sidx1, didx1, rows1, gsem1, isem1))

        def issue_idx(i, b):
            sidx, didx, _, _, isem = bufs[b]
            off = base + i * M
            pltpu.async_copy(src_hbm.at[pl.ds(off, M)], sidx, isem)
            pltpu.async_copy(dst_hbm.at[pl.ds(off, M)], didx, isem)

        def wait_idx(b):
            sidx, didx, _, _, isem = bufs[b]
            pltpu.make_async_copy(src_hbm.at[pl.ds(0, M)], sidx, isem).wait()
            pltpu.make_async_copy(dst_hbm.at[pl.ds(0, M)], didx, isem).wait()

        def issue_gathers(b):
            sidx, _, rows, gsem, _ = bufs[b]
            for g in range(G):
                pltpu.async_copy(
                    table_hbm.at[cb].at[sidx.at[pl.ds(g * K, K)]],
                    rows.at[pl.ds(g * K, K)], gsem)

        def wait_gathers(b):
            _, _, rows, gsem, _ = bufs[b]
            pltpu.make_async_copy(
                table_hbm.at[0].at[pl.ds(0, M)], rows, gsem).wait()

        def compute(b):
            _, didx, rows, _, _ = bufs[b]
            for g in range(M // 16):
                dvec = didx[pl.ds(g * 16, 16)]
                for e in range(16):
                    sel = jnp.full((16,), e, I32)
                    rowv = jnp.take_along_axis(dvec, sel, axis=0)
                    vals = rows[g * 16 + e, pl.ds(0, 16)]
                    plsc.addupdate_scatter(acc, [rowv, iota16], vals)

            @pl.when(cb < DSPLIT)
            def _():
                for g in range(M // 16):
                    dvec = didx[pl.ds(g * 16, 16)]
                    m = (dvec >= dlo) & (dvec < dlo + ND)
                    loc = jnp.where(m, dvec - dlo, 0)
                    plsc.addupdate_scatter(dacc, [loc, zi16], o16, mask=m)

        def _zacc(r, c):
            acc[r, pl.ds(0, 16)] = z16
            return c
        lax.fori_loop(0, N, _zacc, 0)

        @pl.when(cb < DSPLIT)
        def _():
            def _zdeg(r, c):
                dacc[r, pl.ds(0, 16)] = z16
                return c
            lax.fori_loop(0, ND, _zdeg, 0)

        issue_idx(0, 0)
        wait_idx(0)
        issue_gathers(0)
        issue_idx(1, 1)

        def _pair(j, c):
            wait_gathers(0)
            wait_idx(1)
            issue_gathers(1)
            compute(0)

            @pl.when(j < nj - 1)
            def _():
                issue_idx(2 * j + 2, 0)
            wait_gathers(1)

            @pl.when(j < nj - 1)
            def _():
                wait_idx(0)
                issue_gathers(0)
            compute(1)

            @pl.when(j < nj - 1)
            def _():
                issue_idx(2 * j + 3, 1)
            return c

        lax.fori_loop(0, nj, _pair, 0)

        pltpu.sync_copy(acc, sums_out.at[eg, :, pl.ds(cb * 16, 16)])

        @pl.when(cb < DSPLIT)
        def _():
            pltpu.sync_copy(dacc, degs_out.at[eg, cb])

    mesh = plsc.VectorSubcoreMesh(core_axis_name="c", subcore_axis_name="s")
    return pl.kernel(
        body,
        out_type=(jax.ShapeDtypeStruct((EG, N, D), F32),
                  jax.ShapeDtypeStruct((EG, DSPLIT, ND, 16), F32)),
        mesh=mesh,
        compiler_params=pltpu.CompilerParams(
            use_tc_tiling_on_sc=False, needs_layout_passes=False),
        scratch_types=[
            pltpu.VMEM((N, 16), F32),
            pltpu.VMEM((ND, 16), F32),
            pltpu.VMEM((K * G,), I32),
            pltpu.VMEM((K * G,), I32),
            pltpu.VMEM((K * G, 16), F32),
            pltpu.VMEM((K * G,), I32),
            pltpu.VMEM((K * G,), I32),
            pltpu.VMEM((K * G, 16), F32),
            pltpu.SemaphoreType.DMA,
            pltpu.SemaphoreType.DMA,
            pltpu.SemaphoreType.DMA,
            pltpu.SemaphoreType.DMA,
        ],
    )


_sc_seg1 = _make_sc_seg_sum(IN_FEATS, E0, NUM_DST0, K=80, G=5, DSPLIT=4)
_sc_seg2 = _make_sc_seg_sum(NUM_CLASSES, E1, NUM_DST1, K=80, G=5, DSPLIT=4)


def _tc1_body(x_ref, s_ref, d_ref, ws1_ref, wn1_ref, b1_ref, wn2_ref,
              ws2_ref, p_ref, hs_ref):
    s = s_ref[0] + s_ref[1]
    deg = d_ref[0, :, :1] + d_ref[1, :, :1]
    inv = jnp.where(deg > 0.0, 1.0 / jnp.maximum(deg, 1.0), 0.0)
    mean = s * inv
    h = jnp.dot(x_ref[...], ws1_ref[...], preferred_element_type=F32)
    h = h + jnp.dot(mean, wn1_ref[...], preferred_element_type=F32)
    h = jnp.maximum(h + b1_ref[...], 0.0)
    p_ref[...] = jnp.dot(h, wn2_ref[...], preferred_element_type=F32)
    hs_ref[...] = jnp.dot(h, ws2_ref[...], preferred_element_type=F32)


_TC1_BM = 1000

_tc1 = pl.pallas_call(
    _tc1_body,
    grid=(NUM_DST0 // _TC1_BM,),
    in_specs=[
        pl.BlockSpec((_TC1_BM, IN_FEATS), lambda i: (i, 0)),
        pl.BlockSpec((2, _TC1_BM, IN_FEATS), lambda i: (0, i, 0)),
        pl.BlockSpec((2, _TC1_BM, 16), lambda i: (0, i, 0)),
        pl.BlockSpec((IN_FEATS, H_FEATS), lambda i: (0, 0)),
        pl.BlockSpec((IN_FEATS, H_FEATS), lambda i: (0, 0)),
        pl.BlockSpec((1, H_FEATS), lambda i: (0, 0)),
        pl.BlockSpec((H_FEATS, NUM_CLASSES), lambda i: (0, 0)),
        pl.BlockSpec((H_FEATS, NUM_CLASSES), lambda i: (0, 0)),
    ],
    out_specs=[
        pl.BlockSpec((_TC1_BM, NUM_CLASSES), lambda i: (i, 0)),
        pl.BlockSpec((_TC1_BM, NUM_CLASSES), lambda i: (i, 0)),
    ],
    out_shape=[
        jax.ShapeDtypeStruct((NUM_DST0, NUM_CLASSES), F32),
        jax.ShapeDtypeStruct((NUM_DST0, NUM_CLASSES), F32),
    ],
)


def _tc2_body(hs_ref, s_ref, d_ref, b2_ref, o_ref):
    s = jnp.sum(s_ref[...], axis=0)
    deg = jnp.sum(d_ref[...], axis=0)[:, :1]
    inv = jnp.where(deg > 0.0, 1.0 / jnp.maximum(deg, 1.0), 0.0)
    o_ref[...] = hs_ref[...] + s * inv + b2_ref[...]


_tc2 = pl.pallas_call(
    _tc2_body,
    out_shape=jax.ShapeDtypeStruct((NUM_DST1, NUM_CLASSES), F32),
)


def kernel(x, src0, dst0, src1, dst1, W_self1, W_neigh1, b1,
           W_self2, W_neigh2, b2):
    xt = x.reshape(NUM_SRC0, IN_FEATS // 16, 16).transpose(1, 0, 2)
    sums0, degs0 = _sc_seg1(xt, src0, dst0)
    degs0 = degs0.reshape(2, NUM_DST0, 16)
    p, hsall = _tc1(x[:NUM_DST0], sums0, degs0, W_self1, W_neigh1,
                    b1.reshape(1, -1), W_neigh2, W_self2)
    pt = p.reshape(NUM_DST0, NUM_CLASSES // 16, 16).transpose(1, 0, 2)
    sums1, degs1 = _sc_seg2(pt, src1, dst1)
    degs1 = degs1.reshape(8, NUM_DST1, 16)
    out = _tc2(hsall[:NUM_DST1], sums1, degs1, b2.reshape(1, -1))
    return out


# speedup vs baseline: 2.5041x; 1.1917x over previous
import jax
import jax.numpy as jnp
from jax import lax
from jax.experimental import pallas as pl
from jax.experimental.pallas import tpu as pltpu
from jax.experimental.pallas import tpu_sc as plsc

F32 = jnp.float32
I32 = jnp.int32

NUM_SRC0 = 10000
NUM_DST0 = 5000
NUM_DST1 = 2000
E0 = 160000
E1 = 64000
IN_FEATS = 256
H_FEATS = 256
NUM_CLASSES = 64

NC = 2
NS = 16
NW = NC * NS


def _make_sc_seg_sum(D, E, N, K, G, DSPLIT):
    CB = D // 16
    EG = NW // CB
    assert CB * EG == NW and E % EG == 0
    n_per = E // EG
    M = K * G
    assert n_per % M == 0 and K % 8 == 0 and K <= 128
    nsup = n_per // M
    assert nsup % 2 == 0 and nsup >= 4
    nj = nsup // 2
    assert N % DSPLIT == 0 and DSPLIT <= CB
    ND = N // DSPLIT

    def body(table_hbm, src_hbm, dst_hbm, sums_out, degs_out,
             acc, dacc, sidx0, didx0, rows0, sidx1, didx1, rows1,
             gsem0, gsem1, isem0, isem1):
        ci = lax.axis_index("c")
        si = lax.axis_index("s")
        wid = si * NC + ci
        eg = wid // CB
        cb = wid % CB
        base = eg * n_per
        z16 = jnp.zeros((16,), F32)
        o16 = jnp.ones((16,), F32)
        zi16 = jnp.zeros((16,), I32)
        iota16 = lax.iota(I32, 16)
        dlo = cb * ND

        bufs = ((sidx0, didx0, rows0, gsem0, isem0),
                (sidx1, didx1, rows1, gsem1, isem1))

        def issue_idx(i, b):
            sidx, didx, _, _, isem = bufs[b]
            off = base + i * M
            pltpu.async_copy(src_hbm.at[pl.ds(off, M)], sidx, isem)
            pltpu.async_copy(dst_hbm.at[pl.ds(off, M)], didx, isem)

        def wait_idx(b):
            sidx, didx, _, _, isem = bufs[b]
            pltpu.make_async_copy(src_hbm.at[pl.ds(0, M)], sidx, isem).wait()
            pltpu.make_async_copy(dst_hbm.at[pl.ds(0, M)], didx, isem).wait()

        def issue_gathers(b):
            sidx, _, rows, gsem, _ = bufs[b]
            for g in range(G):
                pltpu.async_copy(
                    table_hbm.at[cb].at[sidx.at[pl.ds(g * K, K)]],
                    rows.at[pl.ds(g * K, K)], gsem)

        def wait_gathers(b):
            _, _, rows, gsem, _ = bufs[b]
            pltpu.make_async_copy(
                table_hbm.at[0].at[pl.ds(0, M)], rows, gsem).wait()

        def compute(b):
            _, didx, rows, _, _ = bufs[b]
            for g in range(M // 16):
                dvec = didx[pl.ds(g * 16, 16)]
                a16 = lax.shift_left(dvec, 4)
                for h in range(2):
                    es = range(h * 8, h * 8 + 8)
                    bcs = [jnp.take_along_axis(
                        a16, jnp.full((16,), e, I32), axis=0) for e in es]
                    addrs = [v | iota16 for v in bcs]
                    valss = [rows[g * 16 + e, pl.ds(0, 16)] for e in es]
                    for a, v in zip(addrs, valss):
                        plsc.addupdate_scatter(acc, [a], v)

            @pl.when(cb < DSPLIT)
            def _():
                for g in range(M // 16):
                    dvec = didx[pl.ds(g * 16, 16)]
                    m = (dvec >= dlo) & (dvec < dlo + ND)
                    loc = jnp.where(m, dvec - dlo, 0)
                    plsc.addupdate_scatter(dacc, [loc, zi16], o16, mask=m)

        def _zacc(r, c):
            acc[pl.ds(r * 16, 16)] = z16
            return c
        lax.fori_loop(0, N, _zacc, 0)

        @pl.when(cb < DSPLIT)
        def _():
            def _zdeg(r, c):
                dacc[r, pl.ds(0, 16)] = z16
                return c
            lax.fori_loop(0, ND, _zdeg, 0)

        issue_idx(0, 0)
        wait_idx(0)
        issue_gathers(0)
        issue_idx(1, 1)

        def _pair(j, c):
            wait_gathers(0)
            wait_idx(1)
            issue_gathers(1)
            compute(0)

            @pl.when(j < nj - 1)
            def _():
                issue_idx(2 * j + 2, 0)
            wait_gathers(1)

            @pl.when(j < nj - 1)
            def _():
                wait_idx(0)
                issue_gathers(0)
            compute(1)

            @pl.when(j < nj - 1)
            def _():
                issue_idx(2 * j + 3, 1)
            return c

        lax.fori_loop(0, nj, _pair, 0)

        pltpu.sync_copy(acc, sums_out.at[eg, pl.ds(cb * (N * 16), N * 16)])

        @pl.when(cb < DSPLIT)
        def _():
            pltpu.sync_copy(dacc, degs_out.at[eg, cb])

    mesh = plsc.VectorSubcoreMesh(core_axis_name="c", subcore_axis_name="s")
    return pl.kernel(
        body,
        out_type=(jax.ShapeDtypeStruct((EG, CB * N * 16), F32),
                  jax.ShapeDtypeStruct((EG, DSPLIT, ND, 16), F32)),
        mesh=mesh,
        compiler_params=pltpu.CompilerParams(
            use_tc_tiling_on_sc=False, needs_layout_passes=False),
        scratch_types=[
            pltpu.VMEM((N * 16,), F32),
            pltpu.VMEM((ND, 16), F32),
            pltpu.VMEM((K * G,), I32),
            pltpu.VMEM((K * G,), I32),
            pltpu.VMEM((K * G, 16), F32),
            pltpu.VMEM((K * G,), I32),
            pltpu.VMEM((K * G,), I32),
            pltpu.VMEM((K * G, 16), F32),
            pltpu.SemaphoreType.DMA,
            pltpu.SemaphoreType.DMA,
            pltpu.SemaphoreType.DMA,
            pltpu.SemaphoreType.DMA,
        ],
    )


_sc_seg1 = _make_sc_seg_sum(IN_FEATS, E0, NUM_DST0, K=80, G=5, DSPLIT=4)
_sc_seg2 = _make_sc_seg_sum(NUM_CLASSES, E1, NUM_DST1, K=80, G=5, DSPLIT=4)


def _tc1_body(x_ref, s_ref, d_ref, ws1_ref, wn1_ref, b1_ref, wn2_ref,
              ws2_ref, p_ref, hs_ref):
    s = s_ref[0] + s_ref[1]
    deg = d_ref[0, :, :1] + d_ref[1, :, :1]
    inv = jnp.where(deg > 0.0, 1.0 / jnp.maximum(deg, 1.0), 0.0)
    mean = s * inv
    h = jnp.dot(x_ref[...], ws1_ref[...], preferred_element_type=F32)
    h = h + jnp.dot(mean, wn1_ref[...], preferred_element_type=F32)
    h = jnp.maximum(h + b1_ref[...], 0.0)
    p_ref[...] = jnp.dot(h, wn2_ref[...], preferred_element_type=F32)
    hs_ref[...] = jnp.dot(h, ws2_ref[...], preferred_element_type=F32)


_TC1_BM = 1000

_tc1 = pl.pallas_call(
    _tc1_body,
    grid=(NUM_DST0 // _TC1_BM,),
    in_specs=[
        pl.BlockSpec((_TC1_BM, IN_FEATS), lambda i: (i, 0)),
        pl.BlockSpec((2, _TC1_BM, IN_FEATS), lambda i: (0, i, 0)),
        pl.BlockSpec((2, _TC1_BM, 16), lambda i: (0, i, 0)),
        pl.BlockSpec((IN_FEATS, H_FEATS), lambda i: (0, 0)),
        pl.BlockSpec((IN_FEATS, H_FEATS), lambda i: (0, 0)),
        pl.BlockSpec((1, H_FEATS), lambda i: (0, 0)),
        pl.BlockSpec((H_FEATS, NUM_CLASSES), lambda i: (0, 0)),
        pl.BlockSpec((H_FEATS, NUM_CLASSES), lambda i: (0, 0)),
    ],
    out_specs=[
        pl.BlockSpec((_TC1_BM, NUM_CLASSES), lambda i: (i, 0)),
        pl.BlockSpec((_TC1_BM, NUM_CLASSES), lambda i: (i, 0)),
    ],
    out_shape=[
        jax.ShapeDtypeStruct((NUM_DST0, NUM_CLASSES), F32),
        jax.ShapeDtypeStruct((NUM_DST0, NUM_CLASSES), F32),
    ],
)


def _tc2_body(hs_ref, s_ref, d_ref, b2_ref, o_ref):
    s = jnp.sum(s_ref[...], axis=0)
    deg = jnp.sum(d_ref[...], axis=0)[:, :1]
    inv = jnp.where(deg > 0.0, 1.0 / jnp.maximum(deg, 1.0), 0.0)
    o_ref[...] = hs_ref[...] + s * inv + b2_ref[...]


_tc2 = pl.pallas_call(
    _tc2_body,
    out_shape=jax.ShapeDtypeStruct((NUM_DST1, NUM_CLASSES), F32),
)


def kernel(x, src0, dst0, src1, dst1, W_self1, W_neigh1, b1,
           W_self2, W_neigh2, b2):
    xt = x.reshape(NUM_SRC0, IN_FEATS // 16, 16).transpose(1, 0, 2)
    sums0, degs0 = _sc_seg1(xt, src0, dst0)
    sums0 = (sums0.reshape(2, IN_FEATS // 16, NUM_DST0, 16)
             .transpose(0, 2, 1, 3).reshape(2, NUM_DST0, IN_FEATS))
    degs0 = degs0.reshape(2, NUM_DST0, 16)
    p, hsall = _tc1(x[:NUM_DST0], sums0, degs0, W_self1, W_neigh1,
                    b1.reshape(1, -1), W_neigh2, W_self2)
    pt = p.reshape(NUM_DST0, NUM_CLASSES // 16, 16).transpose(1, 0, 2)
    sums1, degs1 = _sc_seg2(pt, src1, dst1)
    sums1 = (sums1.reshape(8, NUM_CLASSES // 16, NUM_DST1, 16)
             .transpose(0, 2, 1, 3).reshape(8, NUM_DST1, NUM_CLASSES))
    degs1 = degs1.reshape(8, NUM_DST1, 16)
    out = _tc2(hsall[:NUM_DST1], sums1, degs1, b2.reshape(1, -1))
    return out
